# baseline (speedup 1.0000x reference)
import functools

import jax
import jax.numpy as jnp
from jax import lax
from jax.experimental import pallas as pl
from jax.experimental.pallas import tpu as pltpu, tpu_sc as plsc

_CH = 128
_SB = 1024


def _tc_repack(i2eT, eye):
    F, N = i2eT.shape
    BW = 2 * _SB
    nb = -(-N // BW)
    TN = (((0,), (0,)), ((), ()))

    def body(blk_ref, eye_ref, out_ref):
        blk = blk_ref[...]
        e = eye_ref[...]
        out_ref[:, :F] = lax.dot_general(
            blk[:, :_SB], e, TN, preferred_element_type=jnp.float32)
        out_ref[:, F:] = lax.dot_general(
            blk[:, _SB:], e, TN, preferred_element_type=jnp.float32)

    return pl.pallas_call(
        body,
        grid=(nb,),
        in_specs=[
            pl.BlockSpec((F, BW), lambda m: (0, m)),
            pl.BlockSpec((F, F), lambda m: (0, 0)),
        ],
        out_specs=pl.BlockSpec((_SB, 2 * F), lambda m: (m, 0)),
        out_shape=jax.ShapeDtypeStruct((nb * _SB, 2 * F), jnp.float32),
        compiler_params=pltpu.CompilerParams(
            dimension_semantics=("arbitrary",),
            fuse_transposed_lhs_in_matmul=True),
    )(i2eT, eye)


def _sc_pair_gather(w, jidx3):
    NW, n_ch, ch = jidx3.shape
    T = NW * n_ch * ch
    D2 = w.shape[1]
    info = plsc.get_sparse_core_info()
    NC = info.num_cores
    mesh = plsc.VectorSubcoreMesh(core_axis_name="c", subcore_axis_name="s")
    b_per_w = n_ch * ch

    @functools.partial(
        pl.kernel,
        mesh=mesh,
        out_type=jax.ShapeDtypeStruct((T, D2), jnp.float32),
        scratch_types=[
            pltpu.VMEM((n_ch, ch), jnp.int32),
            pltpu.VMEM((ch, D2), jnp.float32),
            pltpu.VMEM((ch, D2), jnp.float32),
            pltpu.SemaphoreType.DMA,
            pltpu.SemaphoreType.DMA,
        ],
    )
    def k(w_hbm, idx_hbm, out_hbm, idx_v, buf_a, buf_b, sem_a, sem_b):
        wid = lax.axis_index("s") * NC + lax.axis_index("c")
        base = wid * b_per_w
        pltpu.sync_copy(idx_hbm.at[wid], idx_v)
        bufs = (buf_a, buf_b)
        sems = (sem_a, sem_b)
        descs = {0: pltpu.async_copy(w_hbm.at[idx_v.at[0]], buf_a, sem_a)}
        for j in range(n_ch):
            if j + 1 < n_ch:
                descs[j + 1] = pltpu.async_copy(
                    w_hbm.at[idx_v.at[j + 1]],
                    bufs[(j + 1) % 2], sems[(j + 1) % 2])
            descs[j].wait()
            pltpu.sync_copy(bufs[j % 2],
                            out_hbm.at[pl.ds(base + j * ch, ch)])

    return k(w, jidx3)


def _tc_uselect(u2eT, nodes2d):
    F, N = u2eT.shape
    Bn = nodes2d.shape[1]
    BK = 2048
    nb = -(-N // BK)
    f32 = jnp.float32

    def body(u_ref, n_ref, out_ref):
        i = pl.program_id(0)

        @pl.when(i == 0)
        def _init():
            out_ref[...] = jnp.zeros((F, Bn), f32)

        tcol = lax.broadcasted_iota(jnp.int32, (BK, Bn), 0) + i * BK
        oh = (tcol == n_ref[...]).astype(f32)
        valid = (lax.broadcasted_iota(jnp.int32, (1, BK), 1) + i * BK) < N
        u = jnp.where(valid, u_ref[...], 0.0)
        out_ref[...] += lax.dot_general(
            u, oh, (((1,), (0,)), ((), ())), preferred_element_type=f32)

    return pl.pallas_call(
        body,
        grid=(nb,),
        in_specs=[
            pl.BlockSpec((F, BK), lambda i: (0, i)),
            pl.BlockSpec((1, Bn), lambda i: (0, 0)),
        ],
        out_specs=pl.BlockSpec((F, Bn), lambda i: (0, 0)),
        out_shape=jax.ShapeDtypeStruct((F, Bn), f32),
        compiler_params=pltpu.CompilerParams(
            dimension_semantics=("arbitrary",)),
    )(u2eT, nodes2d)


def _tc_fused(p2, half2d, rk2d, cu_lo, cu_hi, u_selT, r2e8,
              w1p, w1r, b1c, w2, b2c, a1o, a1u, a1bc, a2, a2bc, a3r, a3b):
    T = p2.shape[0]
    D = p2.shape[1] // 2
    Bn = cu_lo.shape[0]
    BT = 2048
    nb = T // BT
    f32 = jnp.float32
    NT = (((1,), (1,)), ((), ()))
    NN = (((1,), (0,)), ((), ()))

    def dot(a, b, dn):
        return lax.dot_general(a, b, dn, preferred_element_type=f32)

    def body(p_ref, hf_ref, rk_ref, clo_ref, chi_ref, uselT_ref, r2e_ref,
             w1p_ref, w1r_ref, b1_ref, w2_ref, b2_ref,
             a1o_ref, a1u_ref, a1b_ref, a2_ref, a2b_ref, a3_ref, a3b_ref,
             out_ref, m_ref, z_ref):
        i = pl.program_id(0)

        @pl.when(i == 0)
        def _init():
            m_ref[...] = jnp.full((Bn, 1), -1e30, f32)
            z_ref[...] = jnp.zeros((Bn, 1), f32)
            out_ref[...] = jnp.zeros((Bn, D), f32)

        p2b = p_ref[...]
        xp_lo = dot(w1p_ref[...], p2b[:, :D], NT)
        xp_hi = dot(w1p_ref[...], p2b[:, D:], NT)
        xp = jnp.where(hf_ref[...] != 0, xp_hi, xp_lo)
        oh_r = (lax.broadcasted_iota(jnp.int32, (8, BT), 0)
                == rk_ref[...]).astype(f32)
        rbT = dot(w1r_ref[...], r2e_ref[...], NT)
        x = jnp.maximum(xp + dot(rbT, oh_r, NN) + b1_ref[...], 0.0)
        o = jnp.maximum(dot(w2_ref[...], x, NN) + b2_ref[...], 0.0)

        t = lax.broadcasted_iota(jnp.int32, (Bn, BT), 1) + i * BT
        mask = (t >= clo_ref[...]) & (t < chi_ref[...])

        ubT = dot(a1u_ref[...], uselT_ref[...], NN) + a1b_ref[...]
        h = jnp.maximum(dot(a1o_ref[...], o, NN)
                        + dot(ubT, mask.astype(f32), NN), 0.0)
        h2 = jnp.maximum(dot(a2_ref[...], h, NN) + a2b_ref[...], 0.0)
        s = dot(a3_ref[...], h2, NN) + a3b_ref[...]

        sb = jnp.broadcast_to(s, (Bn, BT))
        bm = jnp.max(jnp.where(mask, sb, -1e30), axis=1, keepdims=True)
        m_old = m_ref[...]
        m_new = jnp.maximum(m_old, bm)
        scale = jnp.exp(m_old - m_new)
        e = jnp.where(mask, jnp.exp(sb - m_new), 0.0)
        z_ref[...] = z_ref[...] * scale + jnp.sum(e, axis=1, keepdims=True)
        out_ref[...] = out_ref[...] * scale + dot(e, o, NT)
        m_ref[...] = m_new

        @pl.when(i == nb - 1)
        def _fin():
            z = z_ref[...]
            out_ref[...] = jnp.where(z > 0.0, out_ref[...] / z, 0.0)

    full = lambda shape: pl.BlockSpec(shape, lambda i: (0, 0))
    return pl.pallas_call(
        body,
        grid=(nb,),
        in_specs=[
            pl.BlockSpec((BT, 2 * D), lambda i: (i, 0)),
            pl.BlockSpec((1, BT), lambda i: (0, i)),
            pl.BlockSpec((1, BT), lambda i: (0, i)),
            full((Bn, 1)), full((Bn, 1)),
            full((D, Bn)), full((8, D)),
            full((D, D)), full((D, D)), full((D, 1)),
            full((D, D)), full((D, 1)),
            full((D, D)), full((D, D)), full((D, 1)),
            full((D, D)), full((D, 1)),
            full((1, D)), full((1, 1)),
        ],
        out_specs=pl.BlockSpec((Bn, D), lambda i: (0, 0)),
        out_shape=jax.ShapeDtypeStruct((Bn, D), jnp.float32),
        scratch_shapes=[
            pltpu.VMEM((Bn, 1), f32),
            pltpu.VMEM((Bn, 1), f32),
        ],
        compiler_params=pltpu.CompilerParams(
            dimension_semantics=("arbitrary",)),
    )(p2, half2d, rk2d, cu_lo, cu_hi, u_selT, r2e8,
      w1p, w1r, b1c, w2, b2c, a1o, a1u, a1bc, a2, a2bc, a3r, a3b)


def kernel(nodes, flat_item_idx, flat_rating_idx, cu_seqlens,
           u2e, i2e, r2e, w1_W, w1_b, w2_W, w2_b,
           a1_W, a1_b, a2_W, a2_b, a3_W, a3_b):
    T = flat_item_idx.shape[0]
    Bn = nodes.shape[0]
    D = i2e.shape[1]
    info = plsc.get_sparse_core_info()
    NW = info.num_cores * info.num_subcores
    n_ch = T // (NW * _CH)

    i2eT = jnp.swapaxes(i2e, 0, 1)
    u2eT = jnp.swapaxes(u2e, 0, 1)
    w = _tc_repack(i2eT, jnp.eye(D, dtype=jnp.float32))

    idx = flat_item_idx.astype(jnp.int32)
    jrow = (idx // (2 * _SB)) * _SB + idx % _SB
    half = (idx // _SB) % 2
    jidx3 = jrow.reshape(NW, n_ch, _CH)
    p2 = _sc_pair_gather(w, jidx3)

    nodes32 = nodes.astype(jnp.int32)
    u_selT = _tc_uselect(u2eT, nodes32.reshape(1, Bn))

    cu = cu_seqlens.astype(jnp.int32)
    rk2d = flat_rating_idx.astype(jnp.int32).reshape(1, T)
    half2d = half.reshape(1, T)
    cu_lo = cu[:Bn].reshape(Bn, 1)
    cu_hi = cu[1:Bn + 1].reshape(Bn, 1)
    r2e8 = jnp.pad(r2e, ((0, 8 - r2e.shape[0]), (0, 0)))
    w1p, w1r = w1_W[:, :D], w1_W[:, D:]
    a1o, a1u = a1_W[:, :D], a1_W[:, D:]
    return _tc_fused(
        p2, half2d, rk2d, cu_lo, cu_hi, u_selT, r2e8,
        w1p, w1r, w1_b.reshape(D, 1),
        w2_W, w2_b.reshape(D, 1),
        a1o, a1u, a1_b.reshape(D, 1),
        a2_W, a2_b.reshape(D, 1),
        a3_W.reshape(1, D), a3_b.reshape(1, 1),
    )


# speedup vs baseline: 1.5747x; 1.5747x over previous
import functools

import jax
import jax.numpy as jnp
from jax import lax
from jax.experimental import pallas as pl
from jax.experimental.pallas import tpu as pltpu, tpu_sc as plsc

_CH = 128
_SB = 1024


def _tc_repack(i2eT, eye):
    F, N = i2eT.shape
    G = 4
    BW = G * 2 * _SB
    nsb = -(-N // (2 * _SB))
    nb = -(-nsb // G)
    TN = (((0,), (0,)), ((), ()))

    def body(blk_ref, eye_ref, out_ref):
        blk = blk_ref[...]
        e = eye_ref[...]
        for g in range(G):
            lo = blk[:, g * 2 * _SB:g * 2 * _SB + _SB]
            hi = blk[:, g * 2 * _SB + _SB:(g + 1) * 2 * _SB]
            out_ref[g * _SB:(g + 1) * _SB, :F] = lax.dot_general(
                lo, e, TN, preferred_element_type=jnp.float32)
            out_ref[g * _SB:(g + 1) * _SB, F:] = lax.dot_general(
                hi, e, TN, preferred_element_type=jnp.float32)

    return pl.pallas_call(
        body,
        grid=(nb,),
        in_specs=[
            pl.BlockSpec((F, BW), lambda m: (0, m)),
            pl.BlockSpec((F, F), lambda m: (0, 0)),
        ],
        out_specs=pl.BlockSpec((G * _SB, 2 * F), lambda m: (m, 0)),
        out_shape=jax.ShapeDtypeStruct((nb * G * _SB, 2 * F), jnp.float32),
        compiler_params=pltpu.CompilerParams(
            dimension_semantics=("arbitrary",),
            fuse_transposed_lhs_in_matmul=True),
    )(i2eT, eye)


def _sc_pair_gather(w, jidx3):
    NW, n_ch, ch = jidx3.shape
    T = NW * n_ch * ch
    D2 = w.shape[1]
    info = plsc.get_sparse_core_info()
    NC = info.num_cores
    mesh = plsc.VectorSubcoreMesh(core_axis_name="c", subcore_axis_name="s")
    b_per_w = n_ch * ch

    @functools.partial(
        pl.kernel,
        mesh=mesh,
        out_type=jax.ShapeDtypeStruct((T, D2), jnp.float32),
        scratch_types=[
            pltpu.VMEM((n_ch, ch), jnp.int32),
            pltpu.VMEM((ch, D2), jnp.float32),
            pltpu.VMEM((ch, D2), jnp.float32),
            pltpu.SemaphoreType.DMA,
            pltpu.SemaphoreType.DMA,
        ],
    )
    def k(w_hbm, idx_hbm, out_hbm, idx_v, buf_a, buf_b, sem_a, sem_b):
        wid = lax.axis_index("s") * NC + lax.axis_index("c")
        base = wid * b_per_w
        pltpu.sync_copy(idx_hbm.at[wid], idx_v)
        bufs = (buf_a, buf_b)
        sems = (sem_a, sem_b)
        descs = {0: pltpu.async_copy(w_hbm.at[idx_v.at[0]], buf_a, sem_a)}
        for j in range(n_ch):
            if j + 1 < n_ch:
                descs[j + 1] = pltpu.async_copy(
                    w_hbm.at[idx_v.at[j + 1]],
                    bufs[(j + 1) % 2], sems[(j + 1) % 2])
            descs[j].wait()
            pltpu.sync_copy(bufs[j % 2],
                            out_hbm.at[pl.ds(base + j * ch, ch)])

    return k(w, jidx3)


def _tc_uselect(u2eT, nodes2d):
    F, N = u2eT.shape
    Bn = nodes2d.shape[1]
    BK = 2048
    nb = -(-N // BK)
    f32 = jnp.float32

    def body(u_ref, n_ref, out_ref):
        i = pl.program_id(0)

        @pl.when(i == 0)
        def _init():
            out_ref[...] = jnp.zeros((F, Bn), f32)

        tcol = lax.broadcasted_iota(jnp.int32, (BK, Bn), 0) + i * BK
        oh = (tcol == n_ref[...]).astype(f32)
        valid = (lax.broadcasted_iota(jnp.int32, (1, BK), 1) + i * BK) < N
        u = jnp.where(valid, u_ref[...], 0.0)
        out_ref[...] += lax.dot_general(
            u, oh, (((1,), (0,)), ((), ())), preferred_element_type=f32)

    return pl.pallas_call(
        body,
        grid=(nb,),
        in_specs=[
            pl.BlockSpec((F, BK), lambda i: (0, i)),
            pl.BlockSpec((1, Bn), lambda i: (0, 0)),
        ],
        out_specs=pl.BlockSpec((F, Bn), lambda i: (0, 0)),
        out_shape=jax.ShapeDtypeStruct((F, Bn), f32),
        compiler_params=pltpu.CompilerParams(
            dimension_semantics=("arbitrary",)),
    )(u2eT, nodes2d)


def _tc_fused(p2, half2d, rk2d, cu_lo, cu_hi, u_selT, r2e8,
              w1p, w1r, b1c, w2, b2c, a1o, a1u, a1bc, a2, a2bc, a3r, a3b):
    T = p2.shape[0]
    D = p2.shape[1] // 2
    Bn = cu_lo.shape[0]
    BT = 2048
    nb = T // BT
    f32 = jnp.float32
    NT = (((1,), (1,)), ((), ()))
    NN = (((1,), (0,)), ((), ()))

    def dot(a, b, dn):
        return lax.dot_general(a, b, dn, preferred_element_type=f32)

    def body(p_ref, hf_ref, rk_ref, clo_ref, chi_ref, uselT_ref, r2e_ref,
             w1p_ref, w1r_ref, b1_ref, w2_ref, b2_ref,
             a1o_ref, a1u_ref, a1b_ref, a2_ref, a2b_ref, a3_ref, a3b_ref,
             out_ref, m_ref, z_ref):
        i = pl.program_id(0)

        @pl.when(i == 0)
        def _init():
            m_ref[...] = jnp.full((Bn, 1), -1e30, f32)
            z_ref[...] = jnp.zeros((Bn, 1), f32)
            out_ref[...] = jnp.zeros((Bn, D), f32)

        p2b = p_ref[...]
        xp_lo = dot(w1p_ref[...], p2b[:, :D], NT)
        xp_hi = dot(w1p_ref[...], p2b[:, D:], NT)
        xp = jnp.where(hf_ref[...] != 0, xp_hi, xp_lo)
        oh_r = (lax.broadcasted_iota(jnp.int32, (8, BT), 0)
                == rk_ref[...]).astype(f32)
        rbT = dot(w1r_ref[...], r2e_ref[...], NT)
        x = jnp.maximum(xp + dot(rbT, oh_r, NN) + b1_ref[...], 0.0)
        o = jnp.maximum(dot(w2_ref[...], x, NN) + b2_ref[...], 0.0)

        t = lax.broadcasted_iota(jnp.int32, (Bn, BT), 1) + i * BT
        mask = (t >= clo_ref[...]) & (t < chi_ref[...])

        ubT = dot(a1u_ref[...], uselT_ref[...], NN) + a1b_ref[...]
        h = jnp.maximum(dot(a1o_ref[...], o, NN)
                        + dot(ubT, mask.astype(f32), NN), 0.0)
        h2 = jnp.maximum(dot(a2_ref[...], h, NN) + a2b_ref[...], 0.0)
        s = dot(a3_ref[...], h2, NN) + a3b_ref[...]

        sb = jnp.broadcast_to(s, (Bn, BT))
        bm = jnp.max(jnp.where(mask, sb, -1e30), axis=1, keepdims=True)
        m_old = m_ref[...]
        m_new = jnp.maximum(m_old, bm)
        scale = jnp.exp(m_old - m_new)
        e = jnp.where(mask, jnp.exp(sb - m_new), 0.0)
        z_ref[...] = z_ref[...] * scale + jnp.sum(e, axis=1, keepdims=True)
        out_ref[...] = out_ref[...] * scale + dot(e, o, NT)
        m_ref[...] = m_new

        @pl.when(i == nb - 1)
        def _fin():
            z = z_ref[...]
            out_ref[...] = jnp.where(z > 0.0, out_ref[...] / z, 0.0)

    full = lambda shape: pl.BlockSpec(shape, lambda i: (0, 0))
    return pl.pallas_call(
        body,
        grid=(nb,),
        in_specs=[
            pl.BlockSpec((BT, 2 * D), lambda i: (i, 0)),
            pl.BlockSpec((1, BT), lambda i: (0, i)),
            pl.BlockSpec((1, BT), lambda i: (0, i)),
            full((Bn, 1)), full((Bn, 1)),
            full((D, Bn)), full((8, D)),
            full((D, D)), full((D, D)), full((D, 1)),
            full((D, D)), full((D, 1)),
            full((D, D)), full((D, D)), full((D, 1)),
            full((D, D)), full((D, 1)),
            full((1, D)), full((1, 1)),
        ],
        out_specs=pl.BlockSpec((Bn, D), lambda i: (0, 0)),
        out_shape=jax.ShapeDtypeStruct((Bn, D), jnp.float32),
        scratch_shapes=[
            pltpu.VMEM((Bn, 1), f32),
            pltpu.VMEM((Bn, 1), f32),
        ],
        compiler_params=pltpu.CompilerParams(
            dimension_semantics=("arbitrary",)),
    )(p2, half2d, rk2d, cu_lo, cu_hi, u_selT, r2e8,
      w1p, w1r, b1c, w2, b2c, a1o, a1u, a1bc, a2, a2bc, a3r, a3b)


def kernel(nodes, flat_item_idx, flat_rating_idx, cu_seqlens,
           u2e, i2e, r2e, w1_W, w1_b, w2_W, w2_b,
           a1_W, a1_b, a2_W, a2_b, a3_W, a3_b):
    T = flat_item_idx.shape[0]
    Bn = nodes.shape[0]
    D = i2e.shape[1]
    info = plsc.get_sparse_core_info()
    NW = info.num_cores * info.num_subcores
    n_ch = T // (NW * _CH)

    i2eT = jnp.swapaxes(i2e, 0, 1)
    u2eT = jnp.swapaxes(u2e, 0, 1)
    w = _tc_repack(i2eT, jnp.eye(D, dtype=jnp.float32))

    idx = flat_item_idx.astype(jnp.int32)
    jrow = (idx // (2 * _SB)) * _SB + idx % _SB
    half = (idx // _SB) % 2
    jidx3 = jrow.reshape(NW, n_ch, _CH)
    p2 = _sc_pair_gather(w, jidx3)

    nodes32 = nodes.astype(jnp.int32)
    u_selT = _tc_uselect(u2eT, nodes32.reshape(1, Bn))

    cu = cu_seqlens.astype(jnp.int32)
    rk2d = flat_rating_idx.astype(jnp.int32).reshape(1, T)
    half2d = half.reshape(1, T)
    cu_lo = cu[:Bn].reshape(Bn, 1)
    cu_hi = cu[1:Bn + 1].reshape(Bn, 1)
    r2e8 = jnp.pad(r2e, ((0, 8 - r2e.shape[0]), (0, 0)))
    w1p, w1r = w1_W[:, :D], w1_W[:, D:]
    a1o, a1u = a1_W[:, :D], a1_W[:, D:]
    return _tc_fused(
        p2, half2d, rk2d, cu_lo, cu_hi, u_selT, r2e8,
        w1p, w1r, w1_b.reshape(D, 1),
        w2_W, w2_b.reshape(D, 1),
        a1o, a1u, a1_b.reshape(D, 1),
        a2_W, a2_b.reshape(D, 1),
        a3_W.reshape(1, D), a3_b.reshape(1, 1),
    )


# speedup vs baseline: 1.8297x; 1.1619x over previous
import functools

import jax
import jax.numpy as jnp
from jax import lax
from jax.experimental import pallas as pl
from jax.experimental.pallas import tpu as pltpu, tpu_sc as plsc

_CH = 128
_SB = 1024


def _tc_repack(i2eT, eye):
    F, N = i2eT.shape
    G = 8
    BW = G * 2 * _SB
    nsb = -(-N // (2 * _SB))
    nb = -(-nsb // G)
    TN = (((0,), (0,)), ((), ()))

    def body(blk_ref, eye_ref, out_ref):
        blk = blk_ref[...]
        e = eye_ref[...]
        for g in range(G):
            lo = blk[:, g * 2 * _SB:g * 2 * _SB + _SB]
            hi = blk[:, g * 2 * _SB + _SB:(g + 1) * 2 * _SB]
            out_ref[g * _SB:(g + 1) * _SB, :F] = lax.dot_general(
                lo, e, TN, preferred_element_type=jnp.float32)
            out_ref[g * _SB:(g + 1) * _SB, F:] = lax.dot_general(
                hi, e, TN, preferred_element_type=jnp.float32)

    return pl.pallas_call(
        body,
        grid=(nb,),
        in_specs=[
            pl.BlockSpec((F, BW), lambda m: (0, m)),
            pl.BlockSpec((F, F), lambda m: (0, 0)),
        ],
        out_specs=pl.BlockSpec((G * _SB, 2 * F), lambda m: (m, 0)),
        out_shape=jax.ShapeDtypeStruct((nb * G * _SB, 2 * F), jnp.float32),
        compiler_params=pltpu.CompilerParams(
            dimension_semantics=("arbitrary",),
            fuse_transposed_lhs_in_matmul=True),
    )(i2eT, eye)


def _sc_pair_gather(w, jidx3):
    NW, n_ch, ch = jidx3.shape
    T = NW * n_ch * ch
    D2 = w.shape[1]
    info = plsc.get_sparse_core_info()
    NC = info.num_cores
    mesh = plsc.VectorSubcoreMesh(core_axis_name="c", subcore_axis_name="s")
    b_per_w = n_ch * ch

    @functools.partial(
        pl.kernel,
        mesh=mesh,
        out_type=jax.ShapeDtypeStruct((T, D2), jnp.float32),
        scratch_types=[
            pltpu.VMEM((n_ch, ch), jnp.int32),
            pltpu.VMEM((ch, D2), jnp.float32),
            pltpu.VMEM((ch, D2), jnp.float32),
            pltpu.SemaphoreType.DMA,
            pltpu.SemaphoreType.DMA,
        ],
    )
    def k(w_hbm, idx_hbm, out_hbm, idx_v, buf_a, buf_b, sem_a, sem_b):
        wid = lax.axis_index("s") * NC + lax.axis_index("c")
        base = wid * b_per_w
        pltpu.sync_copy(idx_hbm.at[wid], idx_v)
        bufs = (buf_a, buf_b)
        sems = (sem_a, sem_b)
        descs = {0: pltpu.async_copy(w_hbm.at[idx_v.at[0]], buf_a, sem_a)}
        for j in range(n_ch):
            if j + 1 < n_ch:
                descs[j + 1] = pltpu.async_copy(
                    w_hbm.at[idx_v.at[j + 1]],
                    bufs[(j + 1) % 2], sems[(j + 1) % 2])
            descs[j].wait()
            pltpu.sync_copy(bufs[j % 2],
                            out_hbm.at[pl.ds(base + j * ch, ch)])

    return k(w, jidx3)


def _tc_uselect(u2eT, nodes2d):
    F, N = u2eT.shape
    Bn = nodes2d.shape[1]
    BK = 8192
    nb = -(-N // BK)
    f32 = jnp.float32

    def body(u_ref, n_ref, out_ref):
        i = pl.program_id(0)

        @pl.when(i == 0)
        def _init():
            out_ref[...] = jnp.zeros((F, Bn), f32)

        tcol = lax.broadcasted_iota(jnp.int32, (BK, Bn), 0) + i * BK
        oh = (tcol == n_ref[...]).astype(f32)
        valid = (lax.broadcasted_iota(jnp.int32, (1, BK), 1) + i * BK) < N
        u = jnp.where(valid, u_ref[...], 0.0)
        out_ref[...] += lax.dot_general(
            u, oh, (((1,), (0,)), ((), ())), preferred_element_type=f32)

    return pl.pallas_call(
        body,
        grid=(nb,),
        in_specs=[
            pl.BlockSpec((F, BK), lambda i: (0, i)),
            pl.BlockSpec((1, Bn), lambda i: (0, 0)),
        ],
        out_specs=pl.BlockSpec((F, Bn), lambda i: (0, 0)),
        out_shape=jax.ShapeDtypeStruct((F, Bn), f32),
        compiler_params=pltpu.CompilerParams(
            dimension_semantics=("arbitrary",)),
    )(u2eT, nodes2d)


def _tc_fused(p2, half2d, rk2d, cu_lo, cu_hi, u_selT, r2e8,
              w1p, w1r, b1c, w2, b2c, a1o, a1u, a1bc, a2, a2bc, a3r, a3b):
    T = p2.shape[0]
    D = p2.shape[1] // 2
    Bn = cu_lo.shape[0]
    BT = 2048
    nb = T // BT
    f32 = jnp.float32
    NT = (((1,), (1,)), ((), ()))
    NN = (((1,), (0,)), ((), ()))

    def dot(a, b, dn):
        return lax.dot_general(a, b, dn, preferred_element_type=f32)

    def body(p_ref, hf_ref, rk_ref, clo_ref, chi_ref, uselT_ref, r2e_ref,
             w1p_ref, w1r_ref, b1_ref, w2_ref, b2_ref,
             a1o_ref, a1u_ref, a1b_ref, a2_ref, a2b_ref, a3_ref, a3b_ref,
             out_ref, m_ref, z_ref):
        i = pl.program_id(0)

        @pl.when(i == 0)
        def _init():
            m_ref[...] = jnp.full((Bn, 1), -1e30, f32)
            z_ref[...] = jnp.zeros((Bn, 1), f32)
            out_ref[...] = jnp.zeros((Bn, D), f32)

        p2b = p_ref[...]
        xp_lo = dot(w1p_ref[...], p2b[:, :D], NT)
        xp_hi = dot(w1p_ref[...], p2b[:, D:], NT)
        xp = jnp.where(hf_ref[...] != 0, xp_hi, xp_lo)
        oh_r = (lax.broadcasted_iota(jnp.int32, (8, BT), 0)
                == rk_ref[...]).astype(f32)
        rbT = dot(w1r_ref[...], r2e_ref[...], NT)
        x = jnp.maximum(xp + dot(rbT, oh_r, NN) + b1_ref[...], 0.0)
        o = jnp.maximum(dot(w2_ref[...], x, NN) + b2_ref[...], 0.0)

        t = lax.broadcasted_iota(jnp.int32, (Bn, BT), 1) + i * BT
        mask = (t >= clo_ref[...]) & (t < chi_ref[...])

        ubT = dot(a1u_ref[...], uselT_ref[...], NN) + a1b_ref[...]
        h = jnp.maximum(dot(a1o_ref[...], o, NN)
                        + dot(ubT, mask.astype(f32), NN), 0.0)
        h2 = jnp.maximum(dot(a2_ref[...], h, NN) + a2b_ref[...], 0.0)
        s = dot(a3_ref[...], h2, NN) + a3b_ref[...]

        sb = jnp.broadcast_to(s, (Bn, BT))
        bm = jnp.max(jnp.where(mask, sb, -1e30), axis=1, keepdims=True)
        m_old = m_ref[...]
        m_new = jnp.maximum(m_old, bm)
        scale = jnp.exp(m_old - m_new)
        e = jnp.where(mask, jnp.exp(sb - m_new), 0.0)
        z_ref[...] = z_ref[...] * scale + jnp.sum(e, axis=1, keepdims=True)
        out_ref[...] = out_ref[...] * scale + dot(e, o, NT)
        m_ref[...] = m_new

        @pl.when(i == nb - 1)
        def _fin():
            z = z_ref[...]
            out_ref[...] = jnp.where(z > 0.0, out_ref[...] / z, 0.0)

    full = lambda shape: pl.BlockSpec(shape, lambda i: (0, 0))
    return pl.pallas_call(
        body,
        grid=(nb,),
        in_specs=[
            pl.BlockSpec((BT, 2 * D), lambda i: (i, 0)),
            pl.BlockSpec((1, BT), lambda i: (0, i)),
            pl.BlockSpec((1, BT), lambda i: (0, i)),
            full((Bn, 1)), full((Bn, 1)),
            full((D, Bn)), full((8, D)),
            full((D, D)), full((D, D)), full((D, 1)),
            full((D, D)), full((D, 1)),
            full((D, D)), full((D, D)), full((D, 1)),
            full((D, D)), full((D, 1)),
            full((1, D)), full((1, 1)),
        ],
        out_specs=pl.BlockSpec((Bn, D), lambda i: (0, 0)),
        out_shape=jax.ShapeDtypeStruct((Bn, D), jnp.float32),
        scratch_shapes=[
            pltpu.VMEM((Bn, 1), f32),
            pltpu.VMEM((Bn, 1), f32),
        ],
        compiler_params=pltpu.CompilerParams(
            dimension_semantics=("arbitrary",)),
    )(p2, half2d, rk2d, cu_lo, cu_hi, u_selT, r2e8,
      w1p, w1r, b1c, w2, b2c, a1o, a1u, a1bc, a2, a2bc, a3r, a3b)


def kernel(nodes, flat_item_idx, flat_rating_idx, cu_seqlens,
           u2e, i2e, r2e, w1_W, w1_b, w2_W, w2_b,
           a1_W, a1_b, a2_W, a2_b, a3_W, a3_b):
    T = flat_item_idx.shape[0]
    Bn = nodes.shape[0]
    D = i2e.shape[1]
    info = plsc.get_sparse_core_info()
    NW = info.num_cores * info.num_subcores
    n_ch = T // (NW * _CH)

    i2eT = jnp.swapaxes(i2e, 0, 1)
    u2eT = jnp.swapaxes(u2e, 0, 1)
    w = _tc_repack(i2eT, jnp.eye(D, dtype=jnp.float32))

    idx = flat_item_idx.astype(jnp.int32)
    jrow = (idx // (2 * _SB)) * _SB + idx % _SB
    half = (idx // _SB) % 2
    jidx3 = jrow.reshape(NW, n_ch, _CH)
    p2 = _sc_pair_gather(w, jidx3)

    nodes32 = nodes.astype(jnp.int32)
    u_selT = _tc_uselect(u2eT, nodes32.reshape(1, Bn))

    cu = cu_seqlens.astype(jnp.int32)
    rk2d = flat_rating_idx.astype(jnp.int32).reshape(1, T)
    half2d = half.reshape(1, T)
    cu_lo = cu[:Bn].reshape(Bn, 1)
    cu_hi = cu[1:Bn + 1].reshape(Bn, 1)
    r2e8 = jnp.pad(r2e, ((0, 8 - r2e.shape[0]), (0, 0)))
    w1p, w1r = w1_W[:, :D], w1_W[:, D:]
    a1o, a1u = a1_W[:, :D], a1_W[:, D:]
    return _tc_fused(
        p2, half2d, rk2d, cu_lo, cu_hi, u_selT, r2e8,
        w1p, w1r, w1_b.reshape(D, 1),
        w2_W, w2_b.reshape(D, 1),
        a1o, a1u, a1_b.reshape(D, 1),
        a2_W, a2_b.reshape(D, 1),
        a3_W.reshape(1, D), a3_b.reshape(1, 1),
    )


# speedup vs baseline: 2.0198x; 1.1039x over previous
import functools

import jax
import jax.numpy as jnp
from jax import lax
from jax.experimental import pallas as pl
from jax.experimental.pallas import tpu as pltpu, tpu_sc as plsc

_CH = 128
_SB = 1024


def _pack_bf16_pair(a, b):
    i32 = jnp.int32
    ya = lax.bitcast_convert_type(a, i32)
    yb = lax.bitcast_convert_type(b, i32)
    ta = ya + 0x7FFF + ((ya >> 16) & 1)
    tb = yb + 0x7FFF + ((yb >> 16) & 1)
    packed = (ta & i32(-65536)) | ((tb >> 16) & 0xFFFF)
    return lax.bitcast_convert_type(packed, jnp.float32)


def _tc_repack(i2eT, eye):
    F, N = i2eT.shape
    G = 4
    BW = G * 4 * _SB
    nsb = -(-N // (4 * _SB))
    nb = -(-nsb // G)
    TN = (((0,), (0,)), ((), ()))

    def body(blk_ref, eye_ref, out_ref):
        blk = blk_ref[...]
        e = eye_ref[...]
        for g in range(G):
            q = [lax.dot_general(
                    blk[:, (4 * g + k) * _SB:(4 * g + k + 1) * _SB],
                    e, TN, preferred_element_type=jnp.float32)
                 for k in range(4)]
            rows = slice(g * _SB, (g + 1) * _SB)
            out_ref[rows, :F] = _pack_bf16_pair(q[0], q[1])
            out_ref[rows, F:] = _pack_bf16_pair(q[2], q[3])

    return pl.pallas_call(
        body,
        grid=(nb,),
        in_specs=[
            pl.BlockSpec((F, BW), lambda m: (0, m)),
            pl.BlockSpec((F, F), lambda m: (0, 0)),
        ],
        out_specs=pl.BlockSpec((G * _SB, 2 * F), lambda m: (m, 0)),
        out_shape=jax.ShapeDtypeStruct((nb * G * _SB, 2 * F), jnp.float32),
        compiler_params=pltpu.CompilerParams(
            dimension_semantics=("arbitrary",),
            fuse_transposed_lhs_in_matmul=True),
    )(i2eT, eye)


def _sc_pair_gather(w, jidx3):
    NW, n_ch, ch = jidx3.shape
    T = NW * n_ch * ch
    D2 = w.shape[1]
    info = plsc.get_sparse_core_info()
    NC = info.num_cores
    mesh = plsc.VectorSubcoreMesh(core_axis_name="c", subcore_axis_name="s")
    b_per_w = n_ch * ch

    @functools.partial(
        pl.kernel,
        mesh=mesh,
        out_type=jax.ShapeDtypeStruct((T, D2), jnp.float32),
        scratch_types=[
            pltpu.VMEM((n_ch, ch), jnp.int32),
            pltpu.VMEM((ch, D2), jnp.float32),
            pltpu.VMEM((ch, D2), jnp.float32),
            pltpu.SemaphoreType.DMA,
            pltpu.SemaphoreType.DMA,
        ],
    )
    def k(w_hbm, idx_hbm, out_hbm, idx_v, buf_a, buf_b, sem_a, sem_b):
        wid = lax.axis_index("s") * NC + lax.axis_index("c")
        base = wid * b_per_w
        pltpu.sync_copy(idx_hbm.at[wid], idx_v)
        bufs = (buf_a, buf_b)
        sems = (sem_a, sem_b)
        descs = {0: pltpu.async_copy(w_hbm.at[idx_v.at[0]], buf_a, sem_a)}
        for j in range(n_ch):
            if j + 1 < n_ch:
                descs[j + 1] = pltpu.async_copy(
                    w_hbm.at[idx_v.at[j + 1]],
                    bufs[(j + 1) % 2], sems[(j + 1) % 2])
            descs[j].wait()
            pltpu.sync_copy(bufs[j % 2],
                            out_hbm.at[pl.ds(base + j * ch, ch)])

    return k(w, jidx3)


def _tc_uselect(u2eT, nodes2d):
    F, N = u2eT.shape
    Bn = nodes2d.shape[1]
    BK = 8192
    nb = -(-N // BK)
    f32 = jnp.float32

    def body(u_ref, n_ref, out_ref):
        i = pl.program_id(0)

        @pl.when(i == 0)
        def _init():
            out_ref[...] = jnp.zeros((F, Bn), f32)

        tcol = lax.broadcasted_iota(jnp.int32, (BK, Bn), 0) + i * BK
        oh = (tcol == n_ref[...]).astype(f32)
        valid = (lax.broadcasted_iota(jnp.int32, (1, BK), 1) + i * BK) < N
        u = jnp.where(valid, u_ref[...], 0.0)
        out_ref[...] += lax.dot_general(
            u, oh, (((1,), (0,)), ((), ())), preferred_element_type=f32)

    return pl.pallas_call(
        body,
        grid=(nb,),
        in_specs=[
            pl.BlockSpec((F, BK), lambda i: (0, i)),
            pl.BlockSpec((1, Bn), lambda i: (0, 0)),
        ],
        out_specs=pl.BlockSpec((F, Bn), lambda i: (0, 0)),
        out_shape=jax.ShapeDtypeStruct((F, Bn), f32),
        compiler_params=pltpu.CompilerParams(
            dimension_semantics=("arbitrary",)),
    )(u2eT, nodes2d)


def _tc_fused(p2, half2d, rk2d, cu_lo, cu_hi, u_selT, r2e8,
              w1p, w1r, b1c, w2, b2c, a1o, a1u, a1bc, a2, a2bc, a3r, a3b):
    T = p2.shape[0]
    D = p2.shape[1] // 2
    Bn = cu_lo.shape[0]
    BT = 2048
    nb = T // BT
    f32 = jnp.float32
    NT = (((1,), (1,)), ((), ()))
    NN = (((1,), (0,)), ((), ()))

    def dot(a, b, dn):
        return lax.dot_general(a, b, dn, preferred_element_type=f32)

    def body(p_ref, hf_ref, rk_ref, clo_ref, chi_ref, uselT_ref, r2e_ref,
             w1p_ref, w1r_ref, b1_ref, w2_ref, b2_ref,
             a1o_ref, a1u_ref, a1b_ref, a2_ref, a2b_ref, a3_ref, a3b_ref,
             out_ref, m_ref, z_ref):
        i = pl.program_id(0)

        @pl.when(i == 0)
        def _init():
            m_ref[...] = jnp.full((Bn, 1), -1e30, f32)
            z_ref[...] = jnp.zeros((Bn, 1), f32)
            out_ref[...] = jnp.zeros((Bn, D), f32)

        yi = lax.bitcast_convert_type(p_ref[...], jnp.int32)
        hi = lax.bitcast_convert_type(yi & jnp.int32(-65536), f32)
        lo = lax.bitcast_convert_type(yi << 16, f32)
        xs = [dot(w1p_ref[...], quarter, NT)
              for quarter in (hi[:, :D], lo[:, :D], hi[:, D:], lo[:, D:])]
        qs = hf_ref[...]
        xp = jnp.where(qs < 2,
                       jnp.where(qs == 0, xs[0], xs[1]),
                       jnp.where(qs == 2, xs[2], xs[3]))
        oh_r = (lax.broadcasted_iota(jnp.int32, (8, BT), 0)
                == rk_ref[...]).astype(f32)
        rbT = dot(w1r_ref[...], r2e_ref[...], NT)
        x = jnp.maximum(xp + dot(rbT, oh_r, NN) + b1_ref[...], 0.0)
        o = jnp.maximum(dot(w2_ref[...], x, NN) + b2_ref[...], 0.0)

        t = lax.broadcasted_iota(jnp.int32, (Bn, BT), 1) + i * BT
        mask = (t >= clo_ref[...]) & (t < chi_ref[...])

        ubT = dot(a1u_ref[...], uselT_ref[...], NN) + a1b_ref[...]
        h = jnp.maximum(dot(a1o_ref[...], o, NN)
                        + dot(ubT, mask.astype(f32), NN), 0.0)
        h2 = jnp.maximum(dot(a2_ref[...], h, NN) + a2b_ref[...], 0.0)
        s = dot(a3_ref[...], h2, NN) + a3b_ref[...]

        sb = jnp.broadcast_to(s, (Bn, BT))
        bm = jnp.max(jnp.where(mask, sb, -1e30), axis=1, keepdims=True)
        m_old = m_ref[...]
        m_new = jnp.maximum(m_old, bm)
        scale = jnp.exp(m_old - m_new)
        e = jnp.where(mask, jnp.exp(sb - m_new), 0.0)
        z_ref[...] = z_ref[...] * scale + jnp.sum(e, axis=1, keepdims=True)
        out_ref[...] = out_ref[...] * scale + dot(e, o, NT)
        m_ref[...] = m_new

        @pl.when(i == nb - 1)
        def _fin():
            z = z_ref[...]
            out_ref[...] = jnp.where(z > 0.0, out_ref[...] / z, 0.0)

    full = lambda shape: pl.BlockSpec(shape, lambda i: (0, 0))
    return pl.pallas_call(
        body,
        grid=(nb,),
        in_specs=[
            pl.BlockSpec((BT, 2 * D), lambda i: (i, 0)),
            pl.BlockSpec((1, BT), lambda i: (0, i)),
            pl.BlockSpec((1, BT), lambda i: (0, i)),
            full((Bn, 1)), full((Bn, 1)),
            full((D, Bn)), full((8, D)),
            full((D, D)), full((D, D)), full((D, 1)),
            full((D, D)), full((D, 1)),
            full((D, D)), full((D, D)), full((D, 1)),
            full((D, D)), full((D, 1)),
            full((1, D)), full((1, 1)),
        ],
        out_specs=pl.BlockSpec((Bn, D), lambda i: (0, 0)),
        out_shape=jax.ShapeDtypeStruct((Bn, D), jnp.float32),
        scratch_shapes=[
            pltpu.VMEM((Bn, 1), f32),
            pltpu.VMEM((Bn, 1), f32),
        ],
        compiler_params=pltpu.CompilerParams(
            dimension_semantics=("arbitrary",)),
    )(p2, half2d, rk2d, cu_lo, cu_hi, u_selT, r2e8,
      w1p, w1r, b1c, w2, b2c, a1o, a1u, a1bc, a2, a2bc, a3r, a3b)


def kernel(nodes, flat_item_idx, flat_rating_idx, cu_seqlens,
           u2e, i2e, r2e, w1_W, w1_b, w2_W, w2_b,
           a1_W, a1_b, a2_W, a2_b, a3_W, a3_b):
    T = flat_item_idx.shape[0]
    Bn = nodes.shape[0]
    D = i2e.shape[1]
    info = plsc.get_sparse_core_info()
    NW = info.num_cores * info.num_subcores
    n_ch = T // (NW * _CH)

    i2eT = jnp.swapaxes(i2e, 0, 1)
    u2eT = jnp.swapaxes(u2e, 0, 1)
    w = _tc_repack(i2eT, jnp.eye(D, dtype=jnp.float32))

    idx = flat_item_idx.astype(jnp.int32)
    jrow = (idx // (4 * _SB)) * _SB + idx % _SB
    half = (idx // _SB) % 4
    jidx3 = jrow.reshape(NW, n_ch, _CH)
    p2 = _sc_pair_gather(w, jidx3)

    nodes32 = nodes.astype(jnp.int32)
    u_selT = _tc_uselect(u2eT, nodes32.reshape(1, Bn))

    cu = cu_seqlens.astype(jnp.int32)
    rk2d = flat_rating_idx.astype(jnp.int32).reshape(1, T)
    half2d = half.reshape(1, T)
    cu_lo = cu[:Bn].reshape(Bn, 1)
    cu_hi = cu[1:Bn + 1].reshape(Bn, 1)
    r2e8 = jnp.pad(r2e, ((0, 8 - r2e.shape[0]), (0, 0)))
    w1p, w1r = w1_W[:, :D], w1_W[:, D:]
    a1o, a1u = a1_W[:, :D], a1_W[:, D:]
    return _tc_fused(
        p2, half2d, rk2d, cu_lo, cu_hi, u_selT, r2e8,
        w1p, w1r, w1_b.reshape(D, 1),
        w2_W, w2_b.reshape(D, 1),
        a1o, a1u, a1_b.reshape(D, 1),
        a2_W, a2_b.reshape(D, 1),
        a3_W.reshape(1, D), a3_b.reshape(1, 1),
    )


# speedup vs baseline: 2.0741x; 1.0268x over previous
import functools

import jax
import jax.numpy as jnp
from jax import lax
from jax.experimental import pallas as pl
from jax.experimental.pallas import tpu as pltpu, tpu_sc as plsc

_CH = 128
_SB = 1024


def _pack_bf16_pair(a, b):
    i32 = jnp.int32
    ya = lax.bitcast_convert_type(a, i32)
    yb = lax.bitcast_convert_type(b, i32)
    ta = ya + 0x7FFF + ((ya >> 16) & 1)
    tb = yb + 0x7FFF + ((yb >> 16) & 1)
    packed = (ta & i32(-65536)) | ((tb >> 16) & 0xFFFF)
    return lax.bitcast_convert_type(packed, jnp.float32)


def _tc_repack(i2eT, eye):
    F, N = i2eT.shape
    G = 8
    BW = G * 4 * _SB
    nsb = -(-N // (4 * _SB))
    nb = -(-nsb // G)
    TN = (((0,), (0,)), ((), ()))

    def body(blk_ref, eye_ref, out_ref):
        blk = blk_ref[...]
        e = eye_ref[...]
        for g in range(G):
            q = [lax.dot_general(
                    blk[:, (4 * g + k) * _SB:(4 * g + k + 1) * _SB],
                    e, TN, preferred_element_type=jnp.float32)
                 for k in range(4)]
            rows = slice(g * _SB, (g + 1) * _SB)
            out_ref[rows, :F] = _pack_bf16_pair(q[0], q[1])
            out_ref[rows, F:] = _pack_bf16_pair(q[2], q[3])

    return pl.pallas_call(
        body,
        grid=(nb,),
        in_specs=[
            pl.BlockSpec((F, BW), lambda m: (0, m)),
            pl.BlockSpec((F, F), lambda m: (0, 0)),
        ],
        out_specs=pl.BlockSpec((G * _SB, 2 * F), lambda m: (m, 0)),
        out_shape=jax.ShapeDtypeStruct((nb * G * _SB, 2 * F), jnp.float32),
        compiler_params=pltpu.CompilerParams(
            dimension_semantics=("arbitrary",),
            fuse_transposed_lhs_in_matmul=True),
    )(i2eT, eye)


def _sc_pair_gather(w, jidx3):
    NW, n_ch, ch = jidx3.shape
    T = NW * n_ch * ch
    D2 = w.shape[1]
    info = plsc.get_sparse_core_info()
    NC = info.num_cores
    mesh = plsc.VectorSubcoreMesh(core_axis_name="c", subcore_axis_name="s")
    b_per_w = n_ch * ch

    @functools.partial(
        pl.kernel,
        mesh=mesh,
        out_type=jax.ShapeDtypeStruct((T, D2), jnp.float32),
        scratch_types=[
            pltpu.VMEM((n_ch, ch), jnp.int32),
            pltpu.VMEM((ch, D2), jnp.float32),
            pltpu.VMEM((ch, D2), jnp.float32),
            pltpu.SemaphoreType.DMA,
            pltpu.SemaphoreType.DMA,
        ],
    )
    def k(w_hbm, idx_hbm, out_hbm, idx_v, buf_a, buf_b, sem_a, sem_b):
        wid = lax.axis_index("s") * NC + lax.axis_index("c")
        base = wid * b_per_w
        pltpu.sync_copy(idx_hbm.at[wid], idx_v)
        bufs = (buf_a, buf_b)
        sems = (sem_a, sem_b)
        descs = {0: pltpu.async_copy(w_hbm.at[idx_v.at[0]], buf_a, sem_a)}
        for j in range(n_ch):
            if j + 1 < n_ch:
                descs[j + 1] = pltpu.async_copy(
                    w_hbm.at[idx_v.at[j + 1]],
                    bufs[(j + 1) % 2], sems[(j + 1) % 2])
            descs[j].wait()
            pltpu.sync_copy(bufs[j % 2],
                            out_hbm.at[pl.ds(base + j * ch, ch)])

    return k(w, jidx3)


def _tc_uselect(u2eT, nodes2d):
    F, N = u2eT.shape
    Bn = nodes2d.shape[1]
    BK = 16384
    nb = -(-N // BK)
    f32 = jnp.float32

    def body(u_ref, n_ref, out_ref):
        i = pl.program_id(0)

        @pl.when(i == 0)
        def _init():
            out_ref[...] = jnp.zeros((F, Bn), f32)

        tcol = lax.broadcasted_iota(jnp.int32, (BK, Bn), 0) + i * BK
        oh = (tcol == n_ref[...]).astype(f32)
        valid = (lax.broadcasted_iota(jnp.int32, (1, BK), 1) + i * BK) < N
        u = jnp.where(valid, u_ref[...], 0.0)
        out_ref[...] += lax.dot_general(
            u, oh, (((1,), (0,)), ((), ())), preferred_element_type=f32)

    return pl.pallas_call(
        body,
        grid=(nb,),
        in_specs=[
            pl.BlockSpec((F, BK), lambda i: (0, i)),
            pl.BlockSpec((1, Bn), lambda i: (0, 0)),
        ],
        out_specs=pl.BlockSpec((F, Bn), lambda i: (0, 0)),
        out_shape=jax.ShapeDtypeStruct((F, Bn), f32),
        compiler_params=pltpu.CompilerParams(
            dimension_semantics=("arbitrary",)),
    )(u2eT, nodes2d)


def _tc_fused(p2, half2d, rk2d, cu_lo, cu_hi, u_selT, r2e8,
              w1p, w1r, b1c, w2, b2c, a1o, a1u, a1bc, a2, a2bc, a3r, a3b):
    T = p2.shape[0]
    D = p2.shape[1] // 2
    Bn = cu_lo.shape[0]
    BT = 2048
    nb = T // BT
    f32 = jnp.float32
    NT = (((1,), (1,)), ((), ()))
    NN = (((1,), (0,)), ((), ()))

    def dot(a, b, dn):
        return lax.dot_general(a, b, dn, preferred_element_type=f32)

    def body(p_ref, hf_ref, rk_ref, clo_ref, chi_ref, uselT_ref, r2e_ref,
             w1p_ref, w1r_ref, b1_ref, w2_ref, b2_ref,
             a1o_ref, a1u_ref, a1b_ref, a2_ref, a2b_ref, a3_ref, a3b_ref,
             out_ref, m_ref, z_ref):
        i = pl.program_id(0)

        @pl.when(i == 0)
        def _init():
            m_ref[...] = jnp.full((Bn, 1), -1e30, f32)
            z_ref[...] = jnp.zeros((Bn, 1), f32)
            out_ref[...] = jnp.zeros((Bn, D), f32)

        yi = lax.bitcast_convert_type(p_ref[...], jnp.int32)
        hi = lax.bitcast_convert_type(yi & jnp.int32(-65536), f32)
        lo = lax.bitcast_convert_type(yi << 16, f32)
        xs = [dot(w1p_ref[...], quarter, NT)
              for quarter in (hi[:, :D], lo[:, :D], hi[:, D:], lo[:, D:])]
        qs = hf_ref[...]
        xp = jnp.where(qs < 2,
                       jnp.where(qs == 0, xs[0], xs[1]),
                       jnp.where(qs == 2, xs[2], xs[3]))
        oh_r = (lax.broadcasted_iota(jnp.int32, (8, BT), 0)
                == rk_ref[...]).astype(f32)
        rbT = dot(w1r_ref[...], r2e_ref[...], NT)
        x = jnp.maximum(xp + dot(rbT, oh_r, NN) + b1_ref[...], 0.0)
        o = jnp.maximum(dot(w2_ref[...], x, NN) + b2_ref[...], 0.0)

        t = lax.broadcasted_iota(jnp.int32, (Bn, BT), 1) + i * BT
        mask = (t >= clo_ref[...]) & (t < chi_ref[...])

        ubT = dot(a1u_ref[...], uselT_ref[...], NN) + a1b_ref[...]
        h = jnp.maximum(dot(a1o_ref[...], o, NN)
                        + dot(ubT, mask.astype(f32), NN), 0.0)
        h2 = jnp.maximum(dot(a2_ref[...], h, NN) + a2b_ref[...], 0.0)
        s = dot(a3_ref[...], h2, NN) + a3b_ref[...]

        sb = jnp.broadcast_to(s, (Bn, BT))
        bm = jnp.max(jnp.where(mask, sb, -1e30), axis=1, keepdims=True)
        m_old = m_ref[...]
        m_new = jnp.maximum(m_old, bm)
        scale = jnp.exp(m_old - m_new)
        e = jnp.where(mask, jnp.exp(sb - m_new), 0.0)
        z_ref[...] = z_ref[...] * scale + jnp.sum(e, axis=1, keepdims=True)
        out_ref[...] = out_ref[...] * scale + dot(e, o, NT)
        m_ref[...] = m_new

        @pl.when(i == nb - 1)
        def _fin():
            z = z_ref[...]
            out_ref[...] = jnp.where(z > 0.0, out_ref[...] / z, 0.0)

    full = lambda shape: pl.BlockSpec(shape, lambda i: (0, 0))
    return pl.pallas_call(
        body,
        grid=(nb,),
        in_specs=[
            pl.BlockSpec((BT, 2 * D), lambda i: (i, 0)),
            pl.BlockSpec((1, BT), lambda i: (0, i)),
            pl.BlockSpec((1, BT), lambda i: (0, i)),
            full((Bn, 1)), full((Bn, 1)),
            full((D, Bn)), full((8, D)),
            full((D, D)), full((D, D)), full((D, 1)),
            full((D, D)), full((D, 1)),
            full((D, D)), full((D, D)), full((D, 1)),
            full((D, D)), full((D, 1)),
            full((1, D)), full((1, 1)),
        ],
        out_specs=pl.BlockSpec((Bn, D), lambda i: (0, 0)),
        out_shape=jax.ShapeDtypeStruct((Bn, D), jnp.float32),
        scratch_shapes=[
            pltpu.VMEM((Bn, 1), f32),
            pltpu.VMEM((Bn, 1), f32),
        ],
        compiler_params=pltpu.CompilerParams(
            dimension_semantics=("arbitrary",)),
    )(p2, half2d, rk2d, cu_lo, cu_hi, u_selT, r2e8,
      w1p, w1r, b1c, w2, b2c, a1o, a1u, a1bc, a2, a2bc, a3r, a3b)


def kernel(nodes, flat_item_idx, flat_rating_idx, cu_seqlens,
           u2e, i2e, r2e, w1_W, w1_b, w2_W, w2_b,
           a1_W, a1_b, a2_W, a2_b, a3_W, a3_b):
    T = flat_item_idx.shape[0]
    Bn = nodes.shape[0]
    D = i2e.shape[1]
    info = plsc.get_sparse_core_info()
    NW = info.num_cores * info.num_subcores
    n_ch = T // (NW * _CH)

    i2eT = jnp.swapaxes(i2e, 0, 1)
    u2eT = jnp.swapaxes(u2e, 0, 1)
    w = _tc_repack(i2eT, jnp.eye(D, dtype=jnp.float32))

    idx = flat_item_idx.astype(jnp.int32)
    jrow = (idx // (4 * _SB)) * _SB + idx % _SB
    half = (idx // _SB) % 4
    jidx3 = jrow.reshape(NW, n_ch, _CH)
    p2 = _sc_pair_gather(w, jidx3)

    nodes32 = nodes.astype(jnp.int32)
    u_selT = _tc_uselect(u2eT, nodes32.reshape(1, Bn))

    cu = cu_seqlens.astype(jnp.int32)
    rk2d = flat_rating_idx.astype(jnp.int32).reshape(1, T)
    half2d = half.reshape(1, T)
    cu_lo = cu[:Bn].reshape(Bn, 1)
    cu_hi = cu[1:Bn + 1].reshape(Bn, 1)
    r2e8 = jnp.pad(r2e, ((0, 8 - r2e.shape[0]), (0, 0)))
    w1p, w1r = w1_W[:, :D], w1_W[:, D:]
    a1o, a1u = a1_W[:, :D], a1_W[:, D:]
    return _tc_fused(
        p2, half2d, rk2d, cu_lo, cu_hi, u_selT, r2e8,
        w1p, w1r, w1_b.reshape(D, 1),
        w2_W, w2_b.reshape(D, 1),
        a1o, a1u, a1_b.reshape(D, 1),
        a2_W, a2_b.reshape(D, 1),
        a3_W.reshape(1, D), a3_b.reshape(1, 1),
    )
